# baseline (speedup 1.0000x reference)
import functools

import jax
import jax.numpy as jnp
import numpy as np
from jax import lax
from jax.experimental import pallas as pl
from jax.experimental.pallas import tpu as pltpu
from jax.experimental.pallas import tpu_sc as plsc

XF_N = 193
XF_PAD = 208


L_BASE = 3 * 40
R_BASE = 3 * 94
P_BASE = 3 * 61
OL_BASE = 0
IL_BASE = 3 * 20


def _make_tables():
    srcL = np.zeros(XF_PAD, np.int32)
    srcR = np.zeros(XF_PAD, np.int32)
    sgn = np.ones(XF_PAD, np.float32)
    for q in range(21):
        for d in range(3):
            srcL[3 * q + d] = L_BASE + 3 * q + d
            srcR[3 * q + d] = R_BASE + 3 * q + d
        sgn[3 * q] = -1.0
    for q in range(25):
        for d in range(2):
            srcL[63 + 2 * q + d] = srcR[63 + 2 * q + d] = P_BASE + 3 * q + d
        sgn[63 + 2 * q] = -1.0
    for q in range(20):
        for d in range(2):
            srcL[113 + 2 * q + d] = srcR[113 + 2 * q + d] = OL_BASE + 3 * q + d
            srcL[153 + 2 * q + d] = srcR[153 + 2 * q + d] = IL_BASE + 3 * q + d
        sgn[113 + 2 * q] = -1.0
        sgn[153 + 2 * q] = -1.0

    def pairs(n):
        return np.array([(i, j) for i in range(n) for j in range(i + 1, n)],
                        np.int32)

    def pad(a, n):
        out = np.zeros(n, np.int32)
        out[: len(a)] = a
        return out

    hp, pp, lp = pairs(21), pairs(25), pairs(20)
    tab = np.concatenate([
        srcL, srcR,
        pad(3 * hp[:, 0], 224), pad(3 * hp[:, 1], 224),
        pad(63 + 2 * pp[:, 0], 304), pad(63 + 2 * pp[:, 1], 304),
        pad(113 + 2 * lp[:, 0], 192), pad(113 + 2 * lp[:, 1], 192),
    ])
    assert tab.shape == (1856,)
    return tab, sgn


_TAB_NP, _SGN_NP = _make_tables()
OFF_SRCL, OFF_SRCR = 0, 208
OFF_HPI, OFF_HPJ = 416, 640
OFF_PPI, OFF_PPJ = 864, 1168
OFF_LPI, OFF_LPJ = 1472, 1664

NC, NS = 2, 16
NF = 8
XIN_W = (NF + 1) * 345 + 31 - (((NF + 1) * 345 + 31) % 16)


def _sc_body(xflat_hbm, tab_hbm, sgn_hbm, out_hbm,
             cntv, xin, xfb, obuf, tabv, sgnv, stage, shared, red,
             msrcv, seffv, hacc, cal, car, sem0, sem1, sem2, sem3):
    c = lax.axis_index("c")
    s = lax.axis_index("s")
    wid = s * NC + c
    iota = lax.iota(jnp.int32, 16)
    zf = jnp.zeros((16,), jnp.float32)
    base = jnp.minimum(wid * NF, 200 - NF)

    w0 = base * 345
    a0 = (w0 // 16) * 16
    shift = w0 - a0

    c_cnt = pltpu.make_async_copy(
        xflat_hbm.at[pl.ds(s * (32 * 345), 32 * 345)], cntv, sem0)
    c_tab = pltpu.make_async_copy(tab_hbm, tabv, sem1)
    c_sgn = pltpu.make_async_copy(sgn_hbm, sgnv, sem2)
    c_xin = pltpu.make_async_copy(xflat_hbm.at[pl.ds(a0, XIN_W)], xin, sem3)
    c_cnt.start()
    c_tab.start()
    c_sgn.start()
    c_xin.start()

    c_cnt.wait()
    cal[...] = zf
    car[...] = zf

    @functools.partial(plsc.parallel_loop, 0, 32, unroll=2)
    def _(fr):
        rb = fr * 345
        al = zf
        ar = zf
        for k in range(4):
            lane = iota + 16 * k
            m = lane < 63
            lanec = jnp.minimum(lane, 62)
            vl = plsc.load_gather(cntv, [lanec + (rb + L_BASE)])
            nzl = jnp.where((vl != 0.0) & (vl == vl), 1.0, 0.0)
            vr = plsc.load_gather(cntv, [lanec + (rb + R_BASE)])
            nzr = jnp.where((vr != 0.0) & (vr == vr), 1.0, 0.0)
            if k < 3:
                al = al + nzl
                ar = ar + nzr
            else:
                al = al + jnp.where(m, nzl, 0.0)
                ar = ar + jnp.where(m, nzr, 0.0)
        plsc.addupdate(cal, al)
        plsc.addupdate(car, ar)

    cl = jnp.sum(cal[...])
    cr = jnp.sum(car[...])
    stage[...] = jnp.where(iota == 0, cl, jnp.where(iota == 1, cr, 0.0))
    pltpu.sync_copy(stage, shared.at[s])
    plsc.subcore_barrier()
    pltpu.sync_copy(shared, red)
    tot = zf
    for i in range(NS):
        tot = tot + red[i]
    cl_t = jnp.sum(jnp.where(iota == 0, tot, 0.0))
    cr_t = jnp.sum(jnp.where(iota == 1, tot, 0.0))
    condv = zf + jnp.where(cl_t > cr_t, 1.0, 0.0)
    is_left = condv > 0.0

    c_tab.wait()
    c_sgn.wait()
    for k in range(13):
        lane = iota + 16 * k
        sL = plsc.load_gather(tabv, [lane + OFF_SRCL])
        sR = plsc.load_gather(tabv, [lane + OFF_SRCR])
        msrcv[pl.ds(16 * k, 16)] = jnp.where(is_left, sL, sR)
        sg = plsc.load_gather(sgnv, [lane])
        se = jnp.where(is_left, sg, 1.0)
        seffv[pl.ds(16 * k, 16)] = jnp.where(lane < XF_N, se, 0.0)

    c_xin.wait()
    for fr in range(NF):
        hacc[fr] = zf

    @functools.partial(plsc.parallel_loop, 0, 4, unroll=2)
    def _(k):
        lane = iota + k * 16
        src = plsc.load_gather(msrcv, [lane])
        se = plsc.load_gather(seffv, [lane])
        hm = jnp.where(lane < 63, 1.0, 0.0)
        for fr in range(NF + 1):
            frv = iota * 0 + fr
            v = plsc.load_gather(xin, [src + (shift + fr * 345)])
            v = jnp.where(v == v, v, 0.0)
            if fr < NF:
                plsc.addupdate(hacc.at[fr], v * hm)
            plsc.store_scatter(xfb, [frv, lane], v * se)

    @functools.partial(plsc.parallel_loop, 4, 13, unroll=2)
    def _(k):
        lane = iota + k * 16
        src = plsc.load_gather(msrcv, [lane])
        se = plsc.load_gather(seffv, [lane])
        for fr in range(NF + 1):
            frv = iota * 0 + fr
            v = plsc.load_gather(xin, [src + (shift + fr * 345)])
            v = jnp.where(v == v, v, 0.0)
            plsc.store_scatter(xfb, [frv, lane], v * se)

    def sqrt16(r2):
        r2c = jnp.maximum(r2, jnp.float32(1.1754944e-38))
        i = plsc.bitcast(r2c, jnp.int32)
        y = plsc.bitcast(jnp.int32(0x5F3759DF) - (i >> 1), jnp.float32)
        for _ in range(2):
            y = y * (1.5 - 0.5 * r2c * y * y)
        return r2 * y

    def dist2(frv, pi, pj):
        xi = plsc.load_gather(xfb, [frv, pi])
        yi = plsc.load_gather(xfb, [frv, pi + 1])
        xj = plsc.load_gather(xfb, [frv, pj])
        yj = plsc.load_gather(xfb, [frv, pj + 1])
        dx = xi - xj
        dy = yi - yj
        return dx * dx + dy * dy

    @functools.partial(plsc.parallel_loop, 0, 10, unroll=2)
    def _(k):
        lane = iota + k * 16
        m = lane < 153
        for fr in range(NF):
            frv = iota * 0 + fr
            v = plsc.load_gather(xfb, [frv, lane])
            vn = plsc.load_gather(xfb, [frv + 1, lane])
            plsc.store_scatter(obuf, [frv, lane], v, mask=m)
            plsc.store_scatter(obuf, [frv, lane + 153], v - vn, mask=m)

    @functools.partial(plsc.parallel_loop, 0, 14, unroll=2)
    def _(k):
        lane = iota + k * 16
        m = lane < 210
        pi = plsc.load_gather(tabv, [lane + OFF_HPI])
        pj = plsc.load_gather(tabv, [lane + OFF_HPJ])
        for fr in range(NF):
            frv = iota * 0 + fr
            xi = plsc.load_gather(xfb, [frv, pi])
            yi = plsc.load_gather(xfb, [frv, pi + 1])
            zi = plsc.load_gather(xfb, [frv, pi + 2])
            xj = plsc.load_gather(xfb, [frv, pj])
            yj = plsc.load_gather(xfb, [frv, pj + 1])
            zj = plsc.load_gather(xfb, [frv, pj + 2])
            dx = xi - xj
            dy = yi - yj
            dz = zi - zj
            d = (sqrt16(dx * dx + dy * dy + dz * dz) + 1.0) - 1.0
            plsc.store_scatter(obuf, [frv, lane + 306], d, mask=m)

    @functools.partial(plsc.parallel_loop, 0, 19, unroll=2)
    def _(k):
        lane = iota + k * 16
        m = lane < 300
        pi = plsc.load_gather(tabv, [lane + OFF_PPI])
        pj = plsc.load_gather(tabv, [lane + OFF_PPJ])
        for fr in range(NF):
            frv = iota * 0 + fr
            d = (sqrt16(dist2(frv, pi, pj)) + 1.0) - 1.0
            plsc.store_scatter(obuf, [frv, lane + 516], d, mask=m)

    @functools.partial(plsc.parallel_loop, 0, 12, unroll=2)
    def _(k):
        lane = iota + k * 16
        m = lane < 190
        pi = plsc.load_gather(tabv, [lane + OFF_LPI])
        pj = plsc.load_gather(tabv, [lane + OFF_LPJ])
        for fr in range(NF):
            frv = iota * 0 + fr
            d = (sqrt16(dist2(frv, pi, pj)) + 1.0) - 1.0
            plsc.store_scatter(obuf, [frv, lane + 816], d, mask=m)
            d = (sqrt16(dist2(frv, pi + 40, pj + 40)) + 1.0) - 1.0
            plsc.store_scatter(obuf, [frv, lane + 1006], d, mask=m)

    for fr in range(NF):
        frv = iota * 0 + fr
        hs = jnp.sum(hacc[fr])
        ind = jnp.where(hs != 0.0, 1.0, 0.0)
        vals = ind + jnp.where(iota == 0, 0.0, 1.0)
        plsc.store_scatter(obuf, [frv, iota + 1196], vals, mask=iota < 2)

    pltpu.sync_copy(obuf, out_hbm.at[pl.ds(base, NF)])


@functools.partial(jax.jit, static_argnames=("interpret",))
def _run(packed, interpret=False):
    assert packed.shape == (512 * 345,)
    mesh = plsc.VectorSubcoreMesh(core_axis_name="c", subcore_axis_name="s",
                                  num_cores=NC, num_subcores=NS)
    f = pl.kernel(
        _sc_body,
        out_type=jax.ShapeDtypeStruct((200, 1216), jnp.float32),
        mesh=mesh,
        scratch_types=[
            pltpu.VMEM((32 * 345,), jnp.float32),
            pltpu.VMEM((XIN_W,), jnp.float32),
            pltpu.VMEM((NF + 1, XF_PAD), jnp.float32),
            pltpu.VMEM((NF, 1216), jnp.float32),
            pltpu.VMEM((1856,), jnp.int32),
            pltpu.VMEM((XF_PAD,), jnp.float32),
            pltpu.VMEM((16,), jnp.float32),
            pltpu.VMEM_SHARED((NS, 16), jnp.float32),
            pltpu.VMEM((NS, 16), jnp.float32),
            pltpu.VMEM((XF_PAD,), jnp.int32),
            pltpu.VMEM((XF_PAD,), jnp.float32),
            pltpu.VMEM((NF, 16), jnp.float32),
            pltpu.VMEM((16,), jnp.float32),
            pltpu.VMEM((16,), jnp.float32),
            pltpu.SemaphoreType.DMA,
            pltpu.SemaphoreType.DMA,
            pltpu.SemaphoreType.DMA,
            pltpu.SemaphoreType.DMA,
        ],
        compiler_params=pltpu.CompilerParams(use_tc_tiling_on_sc=False,
                                             needs_layout_passes=False),
        interpret=interpret,
    )
    return f(packed, jnp.asarray(_TAB_NP), jnp.asarray(_SGN_NP))


def kernel(x):
    return _run(x.reshape(512 * 345))[:, :1198].reshape(1, 200, 1198)


# speedup vs baseline: 2.3290x; 2.3290x over previous
import functools

import jax
import jax.numpy as jnp
import numpy as np
from jax import lax
from jax.experimental import pallas as pl
from jax.experimental.pallas import tpu as pltpu
from jax.experimental.pallas import tpu_sc as plsc

XF_N = 193
XF_PAD = 208


def _make_tables():
    srcL = np.zeros(XF_PAD, np.int32)
    srcR = np.zeros(XF_PAD, np.int32)
    sgn = np.ones(XF_PAD, np.float32)
    for q in range(21):
        for d in range(3):
            srcL[3 * q + d] = 3 * q + d
            srcR[3 * q + d] = 63 + 3 * q + d
        sgn[3 * q] = -1.0
    for q in range(25):
        for d in range(2):
            srcL[63 + 2 * q + d] = srcR[63 + 2 * q + d] = 126 + 2 * q + d
        sgn[63 + 2 * q] = -1.0
    for q in range(20):
        for d in range(2):
            srcL[113 + 2 * q + d] = srcR[113 + 2 * q + d] = 176 + 2 * q + d
            srcL[153 + 2 * q + d] = srcR[153 + 2 * q + d] = 216 + 2 * q + d
        sgn[113 + 2 * q] = -1.0
        sgn[153 + 2 * q] = -1.0

    def pairs(n):
        return np.array([(i, j) for i in range(n) for j in range(i + 1, n)],
                        np.int32)

    def pad(a, n):
        out = np.zeros(n, np.int32)
        out[: len(a)] = a
        return out

    hp, pp, lp = pairs(21), pairs(25), pairs(20)
    tab = np.concatenate([
        srcL, srcR,
        pad(3 * hp[:, 0], 224), pad(3 * hp[:, 1], 224),
        pad(63 + 2 * pp[:, 0], 304), pad(63 + 2 * pp[:, 1], 304),
        pad(113 + 2 * lp[:, 0], 192), pad(113 + 2 * lp[:, 1], 192),
    ])
    assert tab.shape == (1856,)
    return tab, sgn


_TAB_NP, _SGN_NP = _make_tables()
OFF_SRCL, OFF_SRCR = 0, 208
OFF_HPI, OFF_HPJ = 416, 640
OFF_PPI, OFF_PPJ = 864, 1168
OFF_LPI, OFF_LPJ = 1472, 1664

NC, NS = 2, 16


def _sc_body(xin_hbm, tab_hbm, sgn_hbm, out_hbm,
             cntv, xin, xfb, obuf, tabv, sgnv, stage, shared, red,
             msrcv, seffv, hacc, cal, car, sem0, sem1, sem2, sem3):
    s = lax.axis_index("s")
    wid = s
    iota = lax.iota(jnp.int32, 16)
    zf = jnp.zeros((16,), jnp.float32)
    base = jnp.minimum(wid * 13, 187)

    c_cnt = pltpu.make_async_copy(xin_hbm.at[pl.ds(s * 32, 32)], cntv, sem0)
    c_tab = pltpu.make_async_copy(tab_hbm, tabv, sem1)
    c_sgn = pltpu.make_async_copy(sgn_hbm, sgnv, sem2)
    c_xin = pltpu.make_async_copy(xin_hbm.at[pl.ds(base, 14)], xin, sem3)
    c_cnt.start()
    c_tab.start()
    c_sgn.start()
    c_xin.start()

    c_cnt.wait()
    cal[...] = zf
    car[...] = zf

    @functools.partial(plsc.parallel_loop, 0, 32, unroll=2)
    def _(fr):
        frv = iota * 0 + fr
        al = zf
        ar = zf
        for k in range(8):
            lane = iota + 16 * k
            v = plsc.load_gather(cntv, [frv, lane])
            nz = jnp.where((v != 0.0) & (v == v), 1.0, 0.0)
            if k < 3:
                al = al + nz
            elif k == 3:
                al = al + jnp.where(lane < 63, nz, 0.0)
                ar = ar + jnp.where(lane >= 63, nz, 0.0)
            elif k < 7:
                ar = ar + nz
            else:
                ar = ar + jnp.where(lane < 126, nz, 0.0)
        plsc.addupdate(cal, al)
        plsc.addupdate(car, ar)

    cl = jnp.sum(cal[...])
    cr = jnp.sum(car[...])
    stage[...] = jnp.where(iota == 0, cl, jnp.where(iota == 1, cr, 0.0))
    pltpu.sync_copy(stage, shared.at[s])
    plsc.subcore_barrier()
    pltpu.sync_copy(shared, red)
    tot = zf
    for i in range(NS):
        tot = tot + red[i]
    cl_t = jnp.sum(jnp.where(iota == 0, tot, 0.0))
    cr_t = jnp.sum(jnp.where(iota == 1, tot, 0.0))
    condv = zf + jnp.where(cl_t > cr_t, 1.0, 0.0)
    is_left = condv > 0.0

    c_tab.wait()
    c_sgn.wait()
    for k in range(13):
        lane = iota + 16 * k
        sL = plsc.load_gather(tabv, [lane + OFF_SRCL])
        sR = plsc.load_gather(tabv, [lane + OFF_SRCR])
        msrcv[pl.ds(16 * k, 16)] = jnp.where(is_left, sL, sR)
        sg = plsc.load_gather(sgnv, [lane])
        se = jnp.where(is_left, sg, 1.0)
        seffv[pl.ds(16 * k, 16)] = jnp.where(lane < XF_N, se, 0.0)

    c_xin.wait()
    for fr in range(13):
        hacc[fr] = zf

    @functools.partial(plsc.parallel_loop, 0, 4, unroll=2)
    def _(k):
        lane = iota + k * 16
        src = plsc.load_gather(msrcv, [lane])
        se = plsc.load_gather(seffv, [lane])
        hm = jnp.where(lane < 63, 1.0, 0.0)
        for fr in range(14):
            frv = iota * 0 + fr
            v = plsc.load_gather(xin, [frv, src])
            v = jnp.where(v == v, v, 0.0)
            if fr < 13:
                plsc.addupdate(hacc.at[fr], v * hm)
            plsc.store_scatter(xfb, [frv, lane], v * se)

    @functools.partial(plsc.parallel_loop, 4, 13, unroll=2)
    def _(k):
        lane = iota + k * 16
        src = plsc.load_gather(msrcv, [lane])
        se = plsc.load_gather(seffv, [lane])
        for fr in range(14):
            frv = iota * 0 + fr
            v = plsc.load_gather(xin, [frv, src])
            v = jnp.where(v == v, v, 0.0)
            plsc.store_scatter(xfb, [frv, lane], v * se)

    def sqrt16(r2):
        r2c = jnp.maximum(r2, jnp.float32(1.1754944e-38))
        i = plsc.bitcast(r2c, jnp.int32)
        y = plsc.bitcast(jnp.int32(0x5F3759DF) - (i >> 1), jnp.float32)
        for _ in range(2):
            y = y * (1.5 - 0.5 * r2c * y * y)
        return r2 * y

    def dist2(frv, pi, pj):
        xi = plsc.load_gather(xfb, [frv, pi])
        yi = plsc.load_gather(xfb, [frv, pi + 1])
        xj = plsc.load_gather(xfb, [frv, pj])
        yj = plsc.load_gather(xfb, [frv, pj + 1])
        dx = xi - xj
        dy = yi - yj
        return dx * dx + dy * dy

    @functools.partial(plsc.parallel_loop, 0, 10, unroll=2)
    def _(k):
        lane = iota + k * 16
        m = lane < 153
        for fr in range(13):
            frv = iota * 0 + fr
            v = plsc.load_gather(xfb, [frv, lane])
            vn = plsc.load_gather(xfb, [frv + 1, lane])
            plsc.store_scatter(obuf, [frv, lane], v, mask=m)
            plsc.store_scatter(obuf, [frv, lane + 153], v - vn, mask=m)

    @functools.partial(plsc.parallel_loop, 0, 14, unroll=2)
    def _(k):
        lane = iota + k * 16
        m = lane < 210
        pi = plsc.load_gather(tabv, [lane + OFF_HPI])
        pj = plsc.load_gather(tabv, [lane + OFF_HPJ])
        for fr in range(13):
            frv = iota * 0 + fr
            xi = plsc.load_gather(xfb, [frv, pi])
            yi = plsc.load_gather(xfb, [frv, pi + 1])
            zi = plsc.load_gather(xfb, [frv, pi + 2])
            xj = plsc.load_gather(xfb, [frv, pj])
            yj = plsc.load_gather(xfb, [frv, pj + 1])
            zj = plsc.load_gather(xfb, [frv, pj + 2])
            dx = xi - xj
            dy = yi - yj
            dz = zi - zj
            d = (sqrt16(dx * dx + dy * dy + dz * dz) + 1.0) - 1.0
            plsc.store_scatter(obuf, [frv, lane + 306], d, mask=m)

    @functools.partial(plsc.parallel_loop, 0, 19, unroll=2)
    def _(k):
        lane = iota + k * 16
        m = lane < 300
        pi = plsc.load_gather(tabv, [lane + OFF_PPI])
        pj = plsc.load_gather(tabv, [lane + OFF_PPJ])
        for fr in range(13):
            frv = iota * 0 + fr
            d = (sqrt16(dist2(frv, pi, pj)) + 1.0) - 1.0
            plsc.store_scatter(obuf, [frv, lane + 516], d, mask=m)

    @functools.partial(plsc.parallel_loop, 0, 12, unroll=2)
    def _(k):
        lane = iota + k * 16
        m = lane < 190
        pi = plsc.load_gather(tabv, [lane + OFF_LPI])
        pj = plsc.load_gather(tabv, [lane + OFF_LPJ])
        for fr in range(13):
            frv = iota * 0 + fr
            d = (sqrt16(dist2(frv, pi, pj)) + 1.0) - 1.0
            plsc.store_scatter(obuf, [frv, lane + 816], d, mask=m)
            d = (sqrt16(dist2(frv, pi + 40, pj + 40)) + 1.0) - 1.0
            plsc.store_scatter(obuf, [frv, lane + 1006], d, mask=m)

    for fr in range(13):
        frv = iota * 0 + fr
        hs = jnp.sum(hacc[fr])
        ind = jnp.where(hs != 0.0, 1.0, 0.0)
        vals = ind + jnp.where(iota == 0, 0.0, 1.0)
        plsc.store_scatter(obuf, [frv, iota + 1196], vals, mask=iota < 2)

    pltpu.sync_copy(obuf, out_hbm.at[pl.ds(base, 13)])


@functools.partial(jax.jit, static_argnames=("interpret",))
def _run(packed, interpret=False):
    mesh = plsc.VectorSubcoreMesh(core_axis_name="c", subcore_axis_name="s",
                                  num_cores=1, num_subcores=NS)
    f = pl.kernel(
        _sc_body,
        out_type=jax.ShapeDtypeStruct((200, 1216), jnp.float32),
        mesh=mesh,
        scratch_types=[
            pltpu.VMEM((32, 256), jnp.float32),
            pltpu.VMEM((14, 256), jnp.float32),
            pltpu.VMEM((14, XF_PAD), jnp.float32),
            pltpu.VMEM((13, 1216), jnp.float32),
            pltpu.VMEM((1856,), jnp.int32),
            pltpu.VMEM((XF_PAD,), jnp.float32),
            pltpu.VMEM((16,), jnp.float32),
            pltpu.VMEM_SHARED((NS, 16), jnp.float32),
            pltpu.VMEM((NS, 16), jnp.float32),
            pltpu.VMEM((XF_PAD,), jnp.int32),
            pltpu.VMEM((XF_PAD,), jnp.float32),
            pltpu.VMEM((13, 16), jnp.float32),
            pltpu.VMEM((16,), jnp.float32),
            pltpu.VMEM((16,), jnp.float32),
            pltpu.SemaphoreType.DMA,
            pltpu.SemaphoreType.DMA,
            pltpu.SemaphoreType.DMA,
            pltpu.SemaphoreType.DMA,
        ],
        compiler_params=pltpu.CompilerParams(use_tc_tiling_on_sc=False,
                                             needs_layout_passes=False),
        interpret=interpret,
    )
    return f(packed, jnp.asarray(_TAB_NP), jnp.asarray(_SGN_NP))


def _prep(x):
    n = x.shape[0]
    return jnp.concatenate([
        x[:, 40:61, :].reshape(n, 63),
        x[:, 94:115, :].reshape(n, 63),
        x[:, 61:86, :2].reshape(n, 50),
        x[:, 0:20, :2].reshape(n, 40),
        x[:, 20:40, :2].reshape(n, 40),
    ], axis=1)


def kernel(x):
    out = _run(_prep(x))
    return out[:, :1198].reshape(1, 200, 1198)
